# Initial kernel scaffold; baseline (speedup 1.0000x reference)
import jax
import jax.numpy as jnp
from jax.experimental import pallas as pl


def kernel(keys, values, importance, params):
    raise NotImplementedError("write your pallas kernel here")


# speedup vs baseline: 4.2017x; 4.2017x over previous
import jax
import jax.numpy as jnp
from jax.experimental import pallas as pl
from jax.experimental.pallas import tpu as pltpu

_EPS = 1e-5


def _ln(h, g, b):
    m = jnp.mean(h, axis=-1, keepdims=True)
    v = jnp.mean((h - m) * (h - m), axis=-1, keepdims=True)
    return (h - m) * jax.lax.rsqrt(v + _EPS) * g + b


def _body(k_ref, v_ref, imp_ref,
          w0e, b0e, g0e, a0e, w1e, b1e, g1e, a1e, w2e, b2e, g2e, a2e,
          w2d, b2d, g2d, a2d, w1d, b1d, g1d, a1d, w0d, b0d, g0d, a0d,
          ck_ref, cv_ref):
    mask = imp_ref[:] >= 0.5

    def pyramid(x):
        h = jnp.dot(x, w0e[:], preferred_element_type=jnp.float32) + b0e[:]
        h = jax.nn.relu(_ln(h, g0e[:], a0e[:]))
        h = jnp.dot(h, w1e[:], preferred_element_type=jnp.float32) + b1e[:]
        h = jax.nn.relu(_ln(h, g1e[:], a1e[:]))
        t = jnp.dot(h, w2e[:], preferred_element_type=jnp.float32) + b2e[:]
        t = jax.nn.relu(_ln(t, g2e[:], a2e[:]))
        o2 = jnp.dot(t, w2d[:], preferred_element_type=jnp.float32) + b2d[:]
        o2 = _ln(o2, g2d[:], a2d[:])
        mid = jnp.where(mask, h, o2)
        o = jnp.dot(mid, w1d[:], preferred_element_type=jnp.float32) + b1d[:]
        o = _ln(o, g1d[:], a1d[:])
        o = jnp.dot(o, w0d[:], preferred_element_type=jnp.float32) + b0d[:]
        o = _ln(o, g0d[:], a0d[:])
        return x + o

    ck_ref[:] = pyramid(k_ref[:])
    cv_ref[:] = pyramid(v_ref[:])


def kernel(keys, values, importance, params):
    B, S, H = keys.shape
    N = B * S
    k2 = keys.reshape(N, H)
    v2 = values.reshape(N, H)
    imp = importance.reshape(N, 1)

    plist = []
    for i in range(3):
        plist += [params['enc_W%d' % i],
                  params['enc_b%d' % i].reshape(1, -1),
                  params['enc_g%d' % i].reshape(1, -1),
                  params['enc_beta%d' % i].reshape(1, -1)]
    for i in (2, 1, 0):
        plist += [params['dec_W%d' % i],
                  params['dec_b%d' % i].reshape(1, -1),
                  params['dec_g%d' % i].reshape(1, -1),
                  params['dec_beta%d' % i].reshape(1, -1)]

    T = 512
    grid = (N // T,)
    row_spec = pl.BlockSpec((T, H), lambda i: (i, 0))
    imp_spec = pl.BlockSpec((T, 1), lambda i: (i, 0))
    param_specs = [pl.BlockSpec(p.shape, lambda i: (0, 0)) for p in plist]

    out = pl.pallas_call(
        _body,
        grid=grid,
        in_specs=[row_spec, row_spec, imp_spec] + param_specs,
        out_specs=[row_spec, row_spec],
        out_shape=[jax.ShapeDtypeStruct((N, H), jnp.float32),
                   jax.ShapeDtypeStruct((N, H), jnp.float32)],
        compiler_params=pltpu.CompilerParams(
            dimension_semantics=("arbitrary",)),
    )(k2, v2, imp, *plist)
    ck, cv = out
    return ck.reshape(B, S, H), cv.reshape(B, S, H)
